# Initial kernel scaffold; baseline (speedup 1.0000x reference)
import jax
import jax.numpy as jnp
from jax.experimental import pallas as pl


def kernel(x, v_raw):
    raise NotImplementedError("write your pallas kernel here")


# speedup vs baseline: 1.0175x; 1.0175x over previous
import jax
import jax.numpy as jnp
import numpy as np
from jax.experimental import pallas as pl

_N = 32
_L = (_N + 1) ** 2
_D = float(np.float32(1.0) / np.float32(32.0))
_STEP = float(np.float32(0.032) / np.float32(32.0))
_R = 256


def _to_f32(s):
    bits = s ^ ((s >> 31) & jnp.int32(0x7FFFFFFF))
    return jax.lax.bitcast_convert_type(bits, jnp.float32)


def _body(x_ref, v_ref, y_ref, ld_ref):
    xb = x_ref[...]
    vb = v_ref[...]
    bits = jax.lax.bitcast_convert_type(vb, jnp.int32)
    smax = jnp.int32(0x7FFFFFFF)
    sign = jnp.int32(-(2 ** 31))
    skey = bits ^ ((bits >> 31) & smax)
    k0 = skey[:, 0, :]
    k1 = skey[:, 1, :]

    x0 = xb[:, 0:1]
    x1 = xb[:, 1:2]
    i0 = jnp.clip((x0 * 32.0).astype(jnp.int32), 0, 31)
    i1 = jnp.clip((x1 * 32.0).astype(jnp.int32), 0, 31)
    rA = i1 * 33 + i0
    rB = (31 - i1) * 33 + i0

    ranks = (rB, rB + 33, rA, rA + 33)
    keys = (k0, k0, k1, k1)

    def step(t, res):
        bit = 31 - t
        one = jnp.left_shift(jnp.int32(1), bit)
        out = []
        for r, k, ru in zip(ranks, keys, res):
            trial_u = ru | one
            t_s = trial_u ^ sign
            cnt = jnp.sum((k < t_s).astype(jnp.int32), axis=1, keepdims=True)
            out.append(jnp.where(cnt <= r, trial_u, ru))
        return tuple(out)

    z = jnp.zeros_like(rA)
    res = jax.lax.fori_loop(0, 32, step, (z, z, z, z))
    sB, sB33, sA, sA33 = (ru ^ sign for ru in res)

    def nxt(k, r, s):
        mmin = jnp.min(jnp.where(k > s, k, smax), axis=1, keepdims=True)
        cle = jnp.sum((k <= s).astype(jnp.int32), axis=1, keepdims=True)
        return jnp.where(cle >= r + 2, s, mmin)

    sB1 = nxt(k0, rB, sB)
    sB34 = nxt(k0, rB + 33, sB33)
    sA1 = nxt(k1, rA, sA)
    sA34 = nxt(k1, rA + 33, sA33)

    f0 = i0.astype(jnp.float32)
    f1 = i1.astype(jnp.float32)
    st = jnp.float32(_STEP)
    lin_i0 = f0 * st
    lin_i0p = (f0 + 1.0) * st
    lin_i1 = f1 * st
    lin_i1p = (f1 + 1.0) * st
    lin_f = (32.0 - f1) * st
    lin_fm = (31.0 - f1) * st

    v0g0 = _to_f32(sB33) + (lin_f + lin_i0)
    v1g0 = _to_f32(sB) + (lin_fm + lin_i0)
    v2g0 = _to_f32(sB34) + (lin_f + lin_i0p)
    v3g0 = _to_f32(sB1) + (lin_fm + lin_i0p)
    v0g1 = _to_f32(sA) + (lin_i1 + lin_i0)
    v1g1 = _to_f32(sA33) + (lin_i1p + lin_i0)
    v2g1 = _to_f32(sA1) + (lin_i1 + lin_i0p)
    v3g1 = _to_f32(sA34) + (lin_i1p + lin_i0p)

    d = jnp.float32(_D)
    bx = f0 * d
    by = f1 * d
    t0 = bx + d - x0
    t1 = by + d - x1
    u0 = x0 - bx
    u1 = x1 - by
    s0 = t0 * t1
    s1 = t0 * u1
    s2 = u0 * t1
    s3 = u0 * u1
    y0 = s0 * v0g0 + s1 * v1g0 + s2 * v2g0 + s3 * v3g0
    y1 = s0 * v0g1 + s1 * v1g1 + s2 * v2g1 + s3 * v3g1
    a = (v2g0 - v0g0) * t1 + (v3g0 - v1g0) * u1
    b_ = (v1g0 - v0g0) * t0 + (v3g0 - v2g0) * u0
    c = (v2g1 - v0g1) * t1 + (v3g1 - v1g1) * u1
    d_ = (v1g1 - v0g1) * t0 + (v3g1 - v2g1) * u0

    y_ref[...] = jnp.concatenate([y0, y1], axis=1)
    ld_ref[...] = jnp.log(a * d_ - b_ * c)


def _run(x, v3, interpret=False):
    batch = x.shape[0]
    return pl.pallas_call(
        _body,
        grid=(batch // _R,),
        in_specs=[
            pl.BlockSpec((_R, 2), lambda i: (i, 0)),
            pl.BlockSpec((_R, 2, _L), lambda i: (i, 0, 0)),
        ],
        out_specs=[
            pl.BlockSpec((_R, 2), lambda i: (i, 0)),
            pl.BlockSpec((_R, 1), lambda i: (i, 0)),
        ],
        out_shape=[
            jax.ShapeDtypeStruct((batch, 2), jnp.float32),
            jax.ShapeDtypeStruct((batch, 1), jnp.float32),
        ],
        interpret=interpret,
    )(x, v3)


def kernel(x, v_raw):
    batch = x.shape[0]
    v3 = v_raw.reshape(batch, 2, _L)
    y, ld = _run(x, v3)
    return (y, ld.reshape(batch))


# speedup vs baseline: 11.2569x; 11.0628x over previous
import jax
import jax.numpy as jnp
import numpy as np
from jax.experimental import pallas as pl

_N = 32
_L = (_N + 1) ** 2
_D = float(np.float32(1.0) / np.float32(32.0))
_STEP = float(np.float32(0.032) / np.float32(32.0))
_CB = 256


def _body(x_ref, v_ref, y_ref, ld_ref):
    sign = jnp.int32(-(2 ** 31))
    smax = jnp.int32(0x7FFFFFFF)

    def tof(s):
        bits = s ^ ((s >> 31) & smax)
        return jax.lax.bitcast_convert_type(bits, jnp.float32)

    k0 = v_ref[0]
    k1 = v_ref[1]
    x0 = x_ref[0:1, :]
    x1 = x_ref[1:2, :]
    i0 = jnp.clip((x0 * 32.0).astype(jnp.int32), 0, 31)
    i1 = jnp.clip((x1 * 32.0).astype(jnp.int32), 0, 31)
    rA = i1 * 33 + i0
    rB = (31 - i1) * 33 + i0

    ranks = (rB, rB + 33, rA, rA + 33)
    keys = (k0, k0, k1, k1)

    def step(t, res):
        bit = 31 - t
        one = jnp.left_shift(jnp.int32(1), bit)
        out = []
        for r, k, ru in zip(ranks, keys, res):
            trial_u = ru | one
            t_f = tof(trial_u ^ sign)
            cnt = jnp.sum((k < t_f).astype(jnp.float32), axis=0, keepdims=True)
            out.append(jnp.where(cnt <= r.astype(jnp.float32), trial_u, ru))
        return tuple(out)

    z = jnp.zeros_like(rA)
    res = jax.lax.fori_loop(0, 32, step, (z, z, z, z))
    fB, fB33, fA, fA33 = (tof(ru ^ sign) for ru in res)

    def nxt(k, r, s):
        mmin = jnp.min(jnp.where(k > s, k, jnp.float32(np.inf)), axis=0, keepdims=True)
        cle = jnp.sum((k <= s).astype(jnp.float32), axis=0, keepdims=True)
        return jnp.where(cle >= (r + 2).astype(jnp.float32), s, mmin)

    fB1 = nxt(k0, rB, fB)
    fB34 = nxt(k0, rB + 33, fB33)
    fA1 = nxt(k1, rA, fA)
    fA34 = nxt(k1, rA + 33, fA33)

    f0 = i0.astype(jnp.float32)
    f1 = i1.astype(jnp.float32)
    st = jnp.float32(_STEP)
    lin_i0 = f0 * st
    lin_i0p = (f0 + 1.0) * st
    lin_i1 = f1 * st
    lin_i1p = (f1 + 1.0) * st
    lin_f = (32.0 - f1) * st
    lin_fm = (31.0 - f1) * st

    v0g0 = fB33 + (lin_f + lin_i0)
    v1g0 = fB + (lin_fm + lin_i0)
    v2g0 = fB34 + (lin_f + lin_i0p)
    v3g0 = fB1 + (lin_fm + lin_i0p)
    v0g1 = fA + (lin_i1 + lin_i0)
    v1g1 = fA33 + (lin_i1p + lin_i0)
    v2g1 = fA1 + (lin_i1 + lin_i0p)
    v3g1 = fA34 + (lin_i1p + lin_i0p)

    d = jnp.float32(_D)
    bx = f0 * d
    by = f1 * d
    t0 = bx + d - x0
    t1 = by + d - x1
    u0 = x0 - bx
    u1 = x1 - by
    s0 = t0 * t1
    s1 = t0 * u1
    s2 = u0 * t1
    s3 = u0 * u1
    y0 = s0 * v0g0 + s1 * v1g0 + s2 * v2g0 + s3 * v3g0
    y1 = s0 * v0g1 + s1 * v1g1 + s2 * v2g1 + s3 * v3g1
    a = (v2g0 - v0g0) * t1 + (v3g0 - v1g0) * u1
    b_ = (v1g0 - v0g0) * t0 + (v3g0 - v2g0) * u0
    c = (v2g1 - v0g1) * t1 + (v3g1 - v1g1) * u1
    d_ = (v1g1 - v0g1) * t0 + (v3g1 - v2g1) * u0

    y_ref[...] = jnp.concatenate([y0, y1], axis=0)
    ld_ref[...] = jnp.log(a * d_ - b_ * c)


def _run(xt, vt, interpret=False):
    batch = xt.shape[1]
    return pl.pallas_call(
        _body,
        grid=(batch // _CB,),
        in_specs=[
            pl.BlockSpec((2, _CB), lambda i: (0, i)),
            pl.BlockSpec((2, _L, _CB), lambda i: (0, 0, i)),
        ],
        out_specs=[
            pl.BlockSpec((2, _CB), lambda i: (0, i)),
            pl.BlockSpec((1, _CB), lambda i: (0, i)),
        ],
        out_shape=[
            jax.ShapeDtypeStruct((2, batch), jnp.float32),
            jax.ShapeDtypeStruct((1, batch), jnp.float32),
        ],
        interpret=interpret,
    )(xt, vt)


def kernel(x, v_raw):
    batch = x.shape[0]
    vt = jnp.transpose(v_raw.reshape(batch, 2, _L), (1, 2, 0))
    yt, ldt = _run(x.T, vt)
    return (yt.T, ldt.reshape(batch))


# speedup vs baseline: 17.7431x; 1.5762x over previous
import jax
import jax.numpy as jnp
import numpy as np
from jax.experimental import pallas as pl

_N = 32
_L = (_N + 1) ** 2
_D = float(np.float32(1.0) / np.float32(32.0))
_STEP = float(np.float32(0.032) / np.float32(32.0))
_CB = 512


def _body(x_ref, v_ref, y_ref, ld_ref):
    sign = jnp.int32(-(2 ** 31))
    smax = jnp.int32(0x7FFFFFFF)

    def tof(s):
        bits = s ^ ((s >> 31) & smax)
        return jax.lax.bitcast_convert_type(bits, jnp.float32)

    k0 = v_ref[0]
    k1 = v_ref[1]
    x0 = x_ref[0:1, :]
    x1 = x_ref[1:2, :]
    i0 = jnp.clip((x0 * 32.0).astype(jnp.int32), 0, 31)
    i1 = jnp.clip((x1 * 32.0).astype(jnp.int32), 0, 31)
    rA = i1 * 33 + i0
    rB = (31 - i1) * 33 + i0

    ranks = (rB, rB + 33, rA, rA + 33)
    keys = (k0, k0, k1, k1)

    def split16(k):
        b = jax.lax.bitcast_convert_type(k, jnp.int32)
        s = b ^ ((b >> 31) & smax)
        hi = (s >> 16).astype(jnp.int16)
        lo = ((s & jnp.int32(0xFFFF)) ^ jnp.int32(0x8000)).astype(jnp.int16)
        return hi, lo

    h0, l0 = split16(k0)
    h1, l1 = split16(k1)
    hkeys = (h0, h0, h1, h1)
    lkeys = (l0, l0, l1, l1)

    def count16(h, t_s):
        m = (h < t_s).astype(jnp.int16)
        acc = m[0:16]
        for j in range(1, 68):
            acc = acc + m[16 * j:16 * (j + 1)]
        cnt = jnp.sum(acc.astype(jnp.int32), axis=0, keepdims=True)
        return cnt + m[1088:1089].astype(jnp.int32)

    def step_hi(t, res):
        bit = 15 - t
        one = jnp.left_shift(jnp.int32(1), bit)
        out = []
        for r, h, ru in zip(ranks, hkeys, res):
            trial_u = ru | one
            t_s = (trial_u ^ jnp.int32(0x8000)).astype(jnp.int16)
            cnt = count16(h, t_s)
            out.append(jnp.where(cnt <= r, trial_u, ru))
        return tuple(out)

    z = jnp.zeros_like(rA)
    res16 = jax.lax.fori_loop(0, 16, step_hi, (z, z, z, z))

    mls, bases = [], []
    sent = jnp.full((1, 1), 32767, jnp.int16)
    for h, l, ru in zip(hkeys, lkeys, res16):
        t_h = (ru ^ jnp.int32(0x8000)).astype(jnp.int16)
        mls.append(jnp.where(h == t_h, l, sent))
        bases.append(count16(h, t_h))

    def step_lo(t, res):
        bit = 15 - t
        one = jnp.left_shift(jnp.int32(1), bit)
        out = []
        for r, ml, base, ru in zip(ranks, mls, bases, res):
            trial = ru | one
            t_s = ((trial & jnp.int32(0xFFFF)) ^ jnp.int32(0x8000)).astype(jnp.int16)
            cnt = base + count16(ml, t_s)
            out.append(jnp.where(cnt <= r, trial, ru))
        return tuple(out)

    res_lo = jax.lax.fori_loop(0, 16, step_lo, (z, z, z, z))
    res = tuple((ru << 16) | rl for ru, rl in zip(res16, res_lo))
    fB, fB33, fA, fA33 = (tof(ru ^ sign) for ru in res)

    def nxt(k, r, s):
        mmin = jnp.min(jnp.where(k > s, k, jnp.float32(np.inf)), axis=0, keepdims=True)
        cle = jnp.sum((k <= s).astype(jnp.float32), axis=0, keepdims=True)
        return jnp.where(cle >= (r + 2).astype(jnp.float32), s, mmin)

    fB1 = nxt(k0, rB, fB)
    fB34 = nxt(k0, rB + 33, fB33)
    fA1 = nxt(k1, rA, fA)
    fA34 = nxt(k1, rA + 33, fA33)

    f0 = i0.astype(jnp.float32)
    f1 = i1.astype(jnp.float32)
    st = jnp.float32(_STEP)
    lin_i0 = f0 * st
    lin_i0p = (f0 + 1.0) * st
    lin_i1 = f1 * st
    lin_i1p = (f1 + 1.0) * st
    lin_f = (32.0 - f1) * st
    lin_fm = (31.0 - f1) * st

    v0g0 = fB33 + (lin_f + lin_i0)
    v1g0 = fB + (lin_fm + lin_i0)
    v2g0 = fB34 + (lin_f + lin_i0p)
    v3g0 = fB1 + (lin_fm + lin_i0p)
    v0g1 = fA + (lin_i1 + lin_i0)
    v1g1 = fA33 + (lin_i1p + lin_i0)
    v2g1 = fA1 + (lin_i1 + lin_i0p)
    v3g1 = fA34 + (lin_i1p + lin_i0p)

    d = jnp.float32(_D)
    bx = f0 * d
    by = f1 * d
    t0 = bx + d - x0
    t1 = by + d - x1
    u0 = x0 - bx
    u1 = x1 - by
    s0 = t0 * t1
    s1 = t0 * u1
    s2 = u0 * t1
    s3 = u0 * u1
    y0 = s0 * v0g0 + s1 * v1g0 + s2 * v2g0 + s3 * v3g0
    y1 = s0 * v0g1 + s1 * v1g1 + s2 * v2g1 + s3 * v3g1
    a = (v2g0 - v0g0) * t1 + (v3g0 - v1g0) * u1
    b_ = (v1g0 - v0g0) * t0 + (v3g0 - v2g0) * u0
    c = (v2g1 - v0g1) * t1 + (v3g1 - v1g1) * u1
    d_ = (v1g1 - v0g1) * t0 + (v3g1 - v2g1) * u0

    y_ref[...] = jnp.concatenate([y0, y1], axis=0)
    ld_ref[...] = jnp.log(a * d_ - b_ * c)


def _run(xt, vt, interpret=False):
    batch = xt.shape[1]
    return pl.pallas_call(
        _body,
        grid=(batch // _CB,),
        in_specs=[
            pl.BlockSpec((2, _CB), lambda i: (0, i)),
            pl.BlockSpec((2, _L, _CB), lambda i: (0, 0, i)),
        ],
        out_specs=[
            pl.BlockSpec((2, _CB), lambda i: (0, i)),
            pl.BlockSpec((1, _CB), lambda i: (0, i)),
        ],
        out_shape=[
            jax.ShapeDtypeStruct((2, batch), jnp.float32),
            jax.ShapeDtypeStruct((1, batch), jnp.float32),
        ],
        interpret=interpret,
    )(xt, vt)


def kernel(x, v_raw):
    batch = x.shape[0]
    vt = jnp.transpose(v_raw.reshape(batch, 2, _L), (1, 2, 0))
    yt, ldt = _run(x.T, vt)
    return (yt.T, ldt.reshape(batch))
